# baseline (speedup 1.0000x reference)
import functools

import jax
import jax.numpy as jnp
from jax import lax
from jax.experimental import pallas as pl
from jax.experimental.pallas import tpu as pltpu
from jax.experimental.pallas import tpu_sc as plsc

ROW_TILE = 64


def _router_body(x_ref, rs_ref, rl_ref, perm_ref, pos_ref, pairs_ref):
    T, D = x_ref.shape
    E = rl_ref.shape[1]
    NT = T // ROW_TILE
    NP = pairs_ref.shape[1]
    x = x_ref[...]
    var = jnp.mean(x * x, axis=1, keepdims=True)
    xn = x * lax.rsqrt(var + 1e-6)
    xn = xn * lax.rsqrt(jnp.float32(D)) * rs_ref[...]
    logits = jnp.dot(xn, rl_ref[...], preferred_element_type=jnp.float32)
    expert = jnp.argmax(logits, axis=1).astype(jnp.int32)

    onehot = (expert[:, None] == lax.broadcasted_iota(jnp.int32, (T, E), 1))
    onehot = onehot.astype(jnp.float32)

    counts = jnp.sum(onehot, axis=0, keepdims=True)
    tri = (lax.broadcasted_iota(jnp.int32, (E, E), 0)
           < lax.broadcasted_iota(jnp.int32, (E, E), 1)).astype(jnp.float32)
    offs = jnp.dot(counts, tri, preferred_element_type=jnp.float32)
    ends = offs + counts

    s = onehot
    k = 1
    while k < T:
        s = s + jnp.concatenate(
            [jnp.zeros((k, E), jnp.float32), s[: T - k, :]], axis=0)
        k *= 2
    rank = jnp.sum(s * onehot, axis=1) - 1.0
    seg_base = jnp.sum(onehot * offs, axis=1)
    pos = rank + seg_base

    pos_i = pos.astype(jnp.int32)
    pos_ref[...] = pos_i

    si = offs.astype(jnp.int32)
    di = ends.astype(jnp.int32)
    nonempty = di > si
    t0 = si // ROW_TILE
    t1 = jnp.where(nonempty, (di - 1) // ROW_TILE, 0)
    nt = jnp.where(nonempty, t1 - t0 + 1, 0)
    ntf = nt.astype(jnp.float32)
    pfx = jnp.dot(ntf, tri, preferred_element_type=jnp.float32)
    pfx_i = pfx.astype(jnp.int32)
    total = jnp.sum(nt)
    eids = lax.broadcasted_iota(jnp.int32, (1, E), 1)
    e_last = jnp.max(jnp.where(nonempty, eids, -1))

    pcol = lax.broadcasted_iota(jnp.int32, (1, NP), 1)
    M = ((pcol >= pfx_i.reshape(E, 1))
         & (pcol < (pfx_i + nt).reshape(E, 1))).astype(jnp.int32)
    pexp = jnp.sum(M * eids.reshape(E, 1), axis=0, keepdims=True)
    ptile = jnp.sum(M * (t0 - pfx_i).reshape(E, 1), axis=0,
                    keepdims=True) + pcol
    pstart = jnp.sum(M * si.reshape(E, 1), axis=0, keepdims=True)
    pend = jnp.sum(M * di.reshape(E, 1), axis=0, keepdims=True)
    live = pcol < total
    pexp = jnp.where(live, pexp, e_last)
    ptile = jnp.where(live, ptile, NT - 1)
    pstart = jnp.where(live, pstart, 0)
    pend = jnp.where(live, pend, 0)
    pairs_ref[...] = jnp.concatenate([ptile, pexp, pstart, pend], axis=0)

    ids = lax.broadcasted_iota(jnp.int32, (1, T), 1).astype(jnp.float32)
    CH = 256
    for j in range(T // CH):
        sel = (pos_i[:, None]
               == (j * CH + lax.broadcasted_iota(jnp.int32, (1, CH), 1)))
        chunk = jnp.dot(ids, sel.astype(jnp.float32),
                        preferred_element_type=jnp.float32)
        perm_ref[pl.ds(j * CH, CH)] = chunk.reshape(CH).astype(jnp.int32)


def _router(x2d, router_scale, router_logits):
    T, D = x2d.shape
    E = router_logits.shape[1]
    NP = T // ROW_TILE + E
    return pl.pallas_call(
        _router_body,
        out_shape=(
            jax.ShapeDtypeStruct((T,), jnp.int32),
            jax.ShapeDtypeStruct((T,), jnp.int32),
            jax.ShapeDtypeStruct((4, NP), jnp.int32),
        ),
    )(x2d, router_scale, router_logits)


def _sc_gather_rows(table, idx):
    T, D = table.shape
    B = idx.shape[0]
    info = plsc.get_sparse_core_info()
    nw = info.num_cores * info.num_subcores
    b_per_w = B // nw
    mesh = plsc.VectorSubcoreMesh(core_axis_name="c", subcore_axis_name="s")

    @functools.partial(
        pl.kernel, mesh=mesh,
        out_type=jax.ShapeDtypeStruct((B, D), jnp.float32),
        scratch_types=[
            pltpu.VMEM((b_per_w,), jnp.int32),
            pltpu.VMEM((b_per_w, D), jnp.float32),
            pltpu.SemaphoreType.DMA,
        ],
    )
    def k(table_hbm, idx_hbm, out_hbm, idx_v, rows_v, sem):
        wid = lax.axis_index("s") * info.num_cores + lax.axis_index("c")
        base = wid * b_per_w
        pltpu.sync_copy(idx_hbm.at[pl.ds(base, b_per_w)], idx_v)
        pltpu.async_copy(table_hbm.at[idx_v], rows_v, sem).wait()
        pltpu.sync_copy(rows_v, out_hbm.at[pl.ds(base, b_per_w)])

    return k(table, idx)


def _ffn_body(pairs_ref, scale_ref, xs_ref, g_ref, l_ref, out_ref):
    i = pl.program_id(0)
    tile_base = pairs_ref[0, i] * ROW_TILE
    start = pairs_ref[2, i]
    end = pairs_ref[3, i]
    sc = scale_ref[pairs_ref[1, i]]

    rows = xs_ref[...]
    w0 = g_ref[0, 0]
    w1 = g_ref[0, 1]
    w2 = l_ref[0]
    dn = (((1,), (1,)), ((), ()))
    g0 = lax.dot_general(rows, w0, dn, preferred_element_type=jnp.float32)
    g1 = lax.dot_general(rows, w1, dn, preferred_element_type=jnp.float32)
    act = jax.nn.gelu(g0) * g1
    o = jnp.dot(act, w2, preferred_element_type=jnp.float32)
    ridx = tile_base + lax.broadcasted_iota(jnp.int32, (ROW_TILE, 1), 0)
    m = (ridx >= start) & (ridx < end)
    out_ref[...] = jnp.where(m, o * sc, out_ref[...])


def _ffn(x_sorted, pairs, gating, linear, scale):
    T, D = x_sorted.shape
    E, _, H, _ = gating.shape
    NP = pairs.shape[1]
    grid_spec = pltpu.PrefetchScalarGridSpec(
        num_scalar_prefetch=1,
        grid=(NP,),
        in_specs=[
            pl.BlockSpec(memory_space=pltpu.SMEM),
            pl.BlockSpec((ROW_TILE, D), lambda i, p: (p[0, i], 0)),
            pl.BlockSpec((1, 2, H, D), lambda i, p: (p[1, i], 0, 0, 0)),
            pl.BlockSpec((1, H, D), lambda i, p: (p[1, i], 0, 0)),
        ],
        out_specs=pl.BlockSpec((ROW_TILE, D), lambda i, p: (p[0, i], 0)),
    )
    return pl.pallas_call(
        _ffn_body,
        grid_spec=grid_spec,
        out_shape=jax.ShapeDtypeStruct((T, D), jnp.float32),
    )(pairs, scale, x_sorted, gating, linear)


def kernel(x, router_scale, router_logits, gating_einsum, linear,
           per_expert_scale):
    B, L, D = x.shape
    x2d = x.reshape(B * L, D)
    perm, pos, pairs = _router(x2d, router_scale, router_logits)
    x_sorted = _sc_gather_rows(x2d, perm)
    out_sorted = _ffn(x_sorted, pairs, gating_einsum, linear,
                      per_expert_scale)
    out = _sc_gather_rows(out_sorted, pos)
    return out.reshape(B, L, D)


# speedup vs baseline: 1.3337x; 1.3337x over previous
import functools

import jax
import jax.numpy as jnp
from jax import lax
from jax.experimental import pallas as pl
from jax.experimental.pallas import tpu as pltpu
from jax.experimental.pallas import tpu_sc as plsc

ROW_TILE = 64
NBUF = 4


def _router_body(x_ref, rs_ref, rl_ref, perm_ref, pos_ref, seg_ref):
    T, D = x_ref.shape
    E = rl_ref.shape[1]
    x = x_ref[...]
    var = jnp.mean(x * x, axis=1, keepdims=True)
    xn = x * lax.rsqrt(var + 1e-6)
    xn = xn * lax.rsqrt(jnp.float32(D)) * rs_ref[...]
    logits = jnp.dot(xn, rl_ref[...], preferred_element_type=jnp.float32)
    expert = jnp.argmax(logits, axis=1).astype(jnp.int32)

    onehot = (expert[:, None] == lax.broadcasted_iota(jnp.int32, (T, E), 1))
    onehot = onehot.astype(jnp.float32)

    counts = jnp.sum(onehot, axis=0, keepdims=True)
    tri = (lax.broadcasted_iota(jnp.int32, (E, E), 0)
           < lax.broadcasted_iota(jnp.int32, (E, E), 1)).astype(jnp.float32)
    offs = jnp.dot(counts, tri, preferred_element_type=jnp.float32)
    ends = offs + counts

    s = onehot
    k = 1
    while k < T:
        s = s + jnp.concatenate(
            [jnp.zeros((k, E), jnp.float32), s[: T - k, :]], axis=0)
        k *= 2
    rank = jnp.sum(s * onehot, axis=1) - 1.0
    seg_base = jnp.sum(onehot * offs, axis=1)
    pos = rank + seg_base

    pos_i = pos.astype(jnp.int32)
    pos_ref[...] = pos_i

    si = offs.astype(jnp.int32)
    di = ends.astype(jnp.int32)
    nonempty = di > si
    t0 = si // ROW_TILE
    nt = jnp.where(nonempty, (di - 1) // ROW_TILE - t0 + 1, 0)
    seg_ref[...] = jnp.concatenate([si, di, t0, nt], axis=0)

    ids = lax.broadcasted_iota(jnp.int32, (1, T), 1).astype(jnp.float32)
    CH = 256
    for j in range(T // CH):
        sel = (pos_i[:, None]
               == (j * CH + lax.broadcasted_iota(jnp.int32, (1, CH), 1)))
        chunk = jnp.dot(ids, sel.astype(jnp.float32),
                        preferred_element_type=jnp.float32)
        perm_ref[pl.ds(j * CH, CH)] = chunk.reshape(CH).astype(jnp.int32)


def _router(x2d, router_scale, router_logits):
    T, D = x2d.shape
    E = router_logits.shape[1]
    return pl.pallas_call(
        _router_body,
        out_shape=(
            jax.ShapeDtypeStruct((T,), jnp.int32),
            jax.ShapeDtypeStruct((T,), jnp.int32),
            jax.ShapeDtypeStruct((4, E), jnp.int32),
        ),
    )(x2d, router_scale, router_logits)


def _sc_gather_rows(table, idx):
    T, D = table.shape
    B = idx.shape[0]
    info = plsc.get_sparse_core_info()
    nw = info.num_cores * info.num_subcores
    b_per_w = B // nw
    mesh = plsc.VectorSubcoreMesh(core_axis_name="c", subcore_axis_name="s")

    @functools.partial(
        pl.kernel, mesh=mesh,
        out_type=jax.ShapeDtypeStruct((B, D), jnp.float32),
        scratch_types=[
            pltpu.VMEM((b_per_w,), jnp.int32),
            pltpu.VMEM((b_per_w, D), jnp.float32),
            pltpu.SemaphoreType.DMA,
        ],
    )
    def k(table_hbm, idx_hbm, out_hbm, idx_v, rows_v, sem):
        wid = lax.axis_index("s") * info.num_cores + lax.axis_index("c")
        base = wid * b_per_w
        pltpu.sync_copy(idx_hbm.at[pl.ds(base, b_per_w)], idx_v)
        pltpu.async_copy(table_hbm.at[idx_v], rows_v, sem).wait()
        pltpu.sync_copy(rows_v, out_hbm.at[pl.ds(base, b_per_w)])

    return k(table, idx)


def _ffn_body(seg_ref, scale_ref, xs_ref, g_hbm, l_hbm, out_ref,
              gbuf, lbuf, gsem, lsem):
    E = g_hbm.shape[0]

    def start_fetch(e, slot):
        pltpu.make_async_copy(g_hbm.at[e], gbuf.at[slot],
                              gsem.at[slot]).start()
        pltpu.make_async_copy(l_hbm.at[e], lbuf.at[slot],
                              lsem.at[slot]).start()

    for b in range(NBUF):
        start_fetch(b, b)

    def expert_body(e, carry):
        slot = lax.rem(e, NBUF)
        pltpu.make_async_copy(g_hbm.at[e], gbuf.at[slot],
                              gsem.at[slot]).wait()
        pltpu.make_async_copy(l_hbm.at[e], lbuf.at[slot],
                              lsem.at[slot]).wait()
        start = seg_ref[0, e]
        end = seg_ref[1, e]
        tile0 = seg_ref[2, e]
        nt = seg_ref[3, e]
        sc = scale_ref[e]
        w0 = gbuf[slot, 0]
        w1 = gbuf[slot, 1]
        w2 = lbuf[slot]

        def tile_body(t, c):
            r0 = (tile0 + t) * ROW_TILE
            rows = xs_ref[pl.ds(r0, ROW_TILE), :]
            dn = (((1,), (1,)), ((), ()))
            g0 = lax.dot_general(rows, w0, dn,
                                 preferred_element_type=jnp.float32)
            g1 = lax.dot_general(rows, w1, dn,
                                 preferred_element_type=jnp.float32)
            act = jax.nn.gelu(g0) * g1
            o = jnp.dot(act, w2, preferred_element_type=jnp.float32)
            ridx = r0 + lax.broadcasted_iota(jnp.int32, (ROW_TILE, 1), 0)
            m = (ridx >= start) & (ridx < end)
            out_ref[pl.ds(r0, ROW_TILE), :] = jnp.where(
                m, o * sc, out_ref[pl.ds(r0, ROW_TILE), :])
            return c

        lax.fori_loop(0, nt, tile_body, 0)

        @pl.when(e + NBUF < E)
        def _():
            start_fetch(e + NBUF, slot)

        return carry

    lax.fori_loop(0, E, expert_body, 0)


def _ffn(x_sorted, seg, gating, linear, scale):
    T, D = x_sorted.shape
    E, _, H, _ = gating.shape
    return pl.pallas_call(
        _ffn_body,
        in_specs=[
            pl.BlockSpec(memory_space=pltpu.SMEM),
            pl.BlockSpec(memory_space=pltpu.SMEM),
            pl.BlockSpec(memory_space=pltpu.VMEM),
            pl.BlockSpec(memory_space=pl.ANY),
            pl.BlockSpec(memory_space=pl.ANY),
        ],
        out_specs=pl.BlockSpec(memory_space=pltpu.VMEM),
        out_shape=jax.ShapeDtypeStruct((T, D), jnp.float32),
        scratch_shapes=[
            pltpu.VMEM((NBUF, 2, H, D), jnp.float32),
            pltpu.VMEM((NBUF, H, D), jnp.float32),
            pltpu.SemaphoreType.DMA((NBUF,)),
            pltpu.SemaphoreType.DMA((NBUF,)),
        ],
    )(seg, scale, x_sorted, gating, linear)


def kernel(x, router_scale, router_logits, gating_einsum, linear,
           per_expert_scale):
    B, L, D = x.shape
    x2d = x.reshape(B * L, D)
    perm, pos, seg = _router(x2d, router_scale, router_logits)
    x_sorted = _sc_gather_rows(x2d, perm)
    out_sorted = _ffn(x_sorted, seg, gating_einsum, linear,
                      per_expert_scale)
    out = _sc_gather_rows(out_sorted, pos)
    return out.reshape(B, L, D)


# speedup vs baseline: 1.3732x; 1.0296x over previous
import functools

import jax
import jax.numpy as jnp
from jax import lax
from jax.experimental import pallas as pl
from jax.experimental.pallas import tpu as pltpu
from jax.experimental.pallas import tpu_sc as plsc

ROW_TILE = 64
NBUF = 4
PCH = 256


def _router_body(x_ref, rs_ref, rl_ref, pos_ref, seg_ref):
    T, D = x_ref.shape
    E = rl_ref.shape[1]
    x = x_ref[...]
    var = jnp.mean(x * x, axis=1, keepdims=True)
    xn = x * lax.rsqrt(var + 1e-6)
    xn = xn * lax.rsqrt(jnp.float32(D)) * rs_ref[...]
    logits = jnp.dot(xn, rl_ref[...], preferred_element_type=jnp.float32)
    expert = jnp.argmax(logits, axis=1).astype(jnp.int32)

    onehot = (expert[:, None] == lax.broadcasted_iota(jnp.int32, (T, E), 1))
    onehot = onehot.astype(jnp.float32)

    counts = jnp.sum(onehot, axis=0, keepdims=True)
    tri = (lax.broadcasted_iota(jnp.int32, (E, E), 0)
           < lax.broadcasted_iota(jnp.int32, (E, E), 1)).astype(jnp.float32)
    offs = jnp.dot(counts, tri, preferred_element_type=jnp.float32)
    ends = offs + counts

    s = onehot
    k = 1
    while k < T:
        s = s + jnp.concatenate(
            [jnp.zeros((k, E), jnp.float32), s[: T - k, :]], axis=0)
        k *= 2
    rank = jnp.sum(s * onehot, axis=1) - 1.0
    seg_base = jnp.sum(onehot * offs, axis=1)
    pos = rank + seg_base
    pos_ref[...] = pos.astype(jnp.int32)

    si = offs.astype(jnp.int32)
    di = ends.astype(jnp.int32)
    nonempty = di > si
    t0 = si // ROW_TILE
    nt = jnp.where(nonempty, (di - 1) // ROW_TILE - t0 + 1, 0)
    seg_ref[...] = jnp.concatenate([si, di, t0, nt], axis=0)


def _router(x2d, router_scale, router_logits):
    T, D = x2d.shape
    E = router_logits.shape[1]
    return pl.pallas_call(
        _router_body,
        out_shape=(
            jax.ShapeDtypeStruct((T,), jnp.int32),
            jax.ShapeDtypeStruct((4, E), jnp.int32),
        ),
    )(x2d, router_scale, router_logits)


def _sc_gather_rows(table, idx):
    T, D = table.shape
    B = idx.shape[0]
    info = plsc.get_sparse_core_info()
    nw = info.num_cores * info.num_subcores
    b_per_w = B // nw
    mesh = plsc.VectorSubcoreMesh(core_axis_name="c", subcore_axis_name="s")

    @functools.partial(
        pl.kernel, mesh=mesh,
        out_type=jax.ShapeDtypeStruct((B, D), jnp.float32),
        scratch_types=[
            pltpu.VMEM((b_per_w,), jnp.int32),
            pltpu.VMEM((b_per_w, D), jnp.float32),
            pltpu.SemaphoreType.DMA,
        ],
    )
    def k(table_hbm, idx_hbm, out_hbm, idx_v, rows_v, sem):
        wid = lax.axis_index("s") * info.num_cores + lax.axis_index("c")
        base = wid * b_per_w
        pltpu.sync_copy(idx_hbm.at[pl.ds(base, b_per_w)], idx_v)
        pltpu.async_copy(table_hbm.at[idx_v], rows_v, sem).wait()
        pltpu.sync_copy(rows_v, out_hbm.at[pl.ds(base, b_per_w)])

    return k(table, idx)


def _ffn_body(seg_ref, scale_ref, pos_ref, x_hbm, g_hbm, l_hbm, out_ref,
              xbuf, xs, gbuf, lbuf, xsem, gsem, lsem):
    E = g_hbm.shape[0]
    T, D = xbuf.shape

    def start_fetch(e, slot):
        pltpu.make_async_copy(g_hbm.at[e], gbuf.at[slot],
                              gsem.at[slot]).start()
        pltpu.make_async_copy(l_hbm.at[e], lbuf.at[slot],
                              lsem.at[slot]).start()

    pltpu.make_async_copy(x_hbm, xbuf, xsem).start()
    for b in range(NBUF):
        start_fetch(b, b)
    pltpu.make_async_copy(x_hbm, xbuf, xsem).wait()

    xb = xbuf[...].astype(jnp.bfloat16)
    pos_col = pos_ref[...][:, None]
    for j in range(T // PCH):
        sel = (pos_col == (j * PCH
                           + lax.broadcasted_iota(jnp.int32, (1, PCH), 1)))
        selb = sel.astype(jnp.bfloat16)
        xs[pl.ds(j * PCH, PCH), :] = lax.dot_general(
            selb, xb, (((0,), (0,)), ((), ())),
            preferred_element_type=jnp.float32)

    def expert_body(e, carry):
        slot = lax.rem(e, NBUF)
        pltpu.make_async_copy(g_hbm.at[e], gbuf.at[slot],
                              gsem.at[slot]).wait()
        pltpu.make_async_copy(l_hbm.at[e], lbuf.at[slot],
                              lsem.at[slot]).wait()
        start = seg_ref[0, e]
        end = seg_ref[1, e]
        tile0 = seg_ref[2, e]
        nt = seg_ref[3, e]
        sc = scale_ref[e]
        w0 = gbuf[slot, 0]
        w1 = gbuf[slot, 1]
        w2 = lbuf[slot]

        def tile_body(t, c):
            r0 = (tile0 + t) * ROW_TILE
            rows = xs[pl.ds(r0, ROW_TILE), :]
            dn = (((1,), (1,)), ((), ()))
            g0 = lax.dot_general(rows, w0, dn,
                                 preferred_element_type=jnp.float32)
            g1 = lax.dot_general(rows, w1, dn,
                                 preferred_element_type=jnp.float32)
            act = jax.nn.gelu(g0) * g1
            o = jnp.dot(act, w2, preferred_element_type=jnp.float32)
            ridx = r0 + lax.broadcasted_iota(jnp.int32, (ROW_TILE, 1), 0)
            m = (ridx >= start) & (ridx < end)
            out_ref[pl.ds(r0, ROW_TILE), :] = jnp.where(
                m, o * sc, out_ref[pl.ds(r0, ROW_TILE), :])
            return c

        lax.fori_loop(0, nt, tile_body, 0)

        @pl.when(e + NBUF < E)
        def _():
            start_fetch(e + NBUF, slot)

        return carry

    lax.fori_loop(0, E, expert_body, 0)


def _ffn(x2d, pos, seg, gating, linear, scale):
    T, D = x2d.shape
    E, _, H, _ = gating.shape
    return pl.pallas_call(
        _ffn_body,
        in_specs=[
            pl.BlockSpec(memory_space=pltpu.SMEM),
            pl.BlockSpec(memory_space=pltpu.SMEM),
            pl.BlockSpec(memory_space=pltpu.VMEM),
            pl.BlockSpec(memory_space=pl.ANY),
            pl.BlockSpec(memory_space=pl.ANY),
            pl.BlockSpec(memory_space=pl.ANY),
        ],
        out_specs=pl.BlockSpec(memory_space=pltpu.VMEM),
        out_shape=jax.ShapeDtypeStruct((T, D), jnp.float32),
        scratch_shapes=[
            pltpu.VMEM((T, D), jnp.float32),
            pltpu.VMEM((T, D), jnp.float32),
            pltpu.VMEM((NBUF, 2, H, D), jnp.float32),
            pltpu.VMEM((NBUF, H, D), jnp.float32),
            pltpu.SemaphoreType.DMA,
            pltpu.SemaphoreType.DMA((NBUF,)),
            pltpu.SemaphoreType.DMA((NBUF,)),
        ],
    )(seg, scale, pos, x2d, gating, linear)


def kernel(x, router_scale, router_logits, gating_einsum, linear,
           per_expert_scale):
    B, L, D = x.shape
    x2d = x.reshape(B * L, D)
    pos, seg = _router(x2d, router_scale, router_logits)
    out_sorted = _ffn(x2d, pos, seg, gating_einsum, linear,
                      per_expert_scale)
    out = _sc_gather_rows(out_sorted, pos)
    return out.reshape(B, L, D)


# speedup vs baseline: 1.4390x; 1.0480x over previous
import functools

import jax
import jax.numpy as jnp
from jax import lax
from jax.experimental import pallas as pl
from jax.experimental.pallas import tpu as pltpu
from jax.experimental.pallas import tpu_sc as plsc

ROW_TILE = 64
NBUF = 4
PCH = 256


def _sc_gather_rows(table, idx):
    T, D = table.shape
    B = idx.shape[0]
    info = plsc.get_sparse_core_info()
    nw = info.num_cores * info.num_subcores
    b_per_w = B // nw
    mesh = plsc.VectorSubcoreMesh(core_axis_name="c", subcore_axis_name="s")

    @functools.partial(
        pl.kernel, mesh=mesh,
        out_type=jax.ShapeDtypeStruct((B, D), jnp.float32),
        scratch_types=[
            pltpu.VMEM((b_per_w,), jnp.int32),
            pltpu.VMEM((b_per_w, D), jnp.float32),
            pltpu.SemaphoreType.DMA,
        ],
    )
    def k(table_hbm, idx_hbm, out_hbm, idx_v, rows_v, sem):
        wid = lax.axis_index("s") * info.num_cores + lax.axis_index("c")
        base = wid * b_per_w
        pltpu.sync_copy(idx_hbm.at[pl.ds(base, b_per_w)], idx_v)
        pltpu.async_copy(table_hbm.at[idx_v], rows_v, sem).wait()
        pltpu.sync_copy(rows_v, out_hbm.at[pl.ds(base, b_per_w)])

    return k(table, idx)


def _moe_body(scale_ref, rs_ref, rl_ref, x_hbm, g_hbm, l_hbm,
              out_ref, pos_ref,
              xbuf, xs, segv, gbuf, lbuf, seg_smem,
              xsem, ssem, gsem, lsem):
    E = g_hbm.shape[0]
    T, D = xbuf.shape

    def start_fetch(e, slot):
        pltpu.make_async_copy(g_hbm.at[e], gbuf.at[slot],
                              gsem.at[slot]).start()
        pltpu.make_async_copy(l_hbm.at[e], lbuf.at[slot],
                              lsem.at[slot]).start()

    pltpu.make_async_copy(x_hbm, xbuf, xsem).start()
    for b in range(NBUF):
        start_fetch(b, b)
    pltpu.make_async_copy(x_hbm, xbuf, xsem).wait()

    x = xbuf[...]
    var = jnp.mean(x * x, axis=1, keepdims=True)
    xn = x * lax.rsqrt(var + 1e-6)
    xn = xn * lax.rsqrt(jnp.float32(D)) * rs_ref[...]
    logits = jnp.dot(xn, rl_ref[...], preferred_element_type=jnp.float32)
    expert = jnp.argmax(logits, axis=1).astype(jnp.int32)

    onehot = (expert[:, None] == lax.broadcasted_iota(jnp.int32, (T, E), 1))
    onehot = onehot.astype(jnp.float32)

    counts = jnp.sum(onehot, axis=0, keepdims=True)
    tri = (lax.broadcasted_iota(jnp.int32, (E, E), 0)
           < lax.broadcasted_iota(jnp.int32, (E, E), 1)).astype(jnp.float32)
    offs = jnp.dot(counts, tri, preferred_element_type=jnp.float32)
    ends = offs + counts

    s = onehot
    k = 1
    while k < T:
        s = s + jnp.concatenate(
            [jnp.zeros((k, E), jnp.float32), s[: T - k, :]], axis=0)
        k *= 2
    rank = jnp.sum(s * onehot, axis=1) - 1.0
    seg_base = jnp.sum(onehot * offs, axis=1)
    pos = rank + seg_base
    pos_i = pos.astype(jnp.int32)
    pos_ref[...] = pos_i

    si = offs.astype(jnp.int32)
    di = ends.astype(jnp.int32)
    nonempty = di > si
    t0 = si // ROW_TILE
    nt = jnp.where(nonempty, (di - 1) // ROW_TILE - t0 + 1, 0)
    segv[...] = jnp.concatenate([si, di, t0, nt], axis=0)
    pltpu.make_async_copy(segv, seg_smem, ssem).start()

    xb = x.astype(jnp.bfloat16)
    pos_col = pos_i[:, None]
    for j in range(T // PCH):
        sel = (pos_col == (j * PCH
                           + lax.broadcasted_iota(jnp.int32, (1, PCH), 1)))
        selb = sel.astype(jnp.bfloat16)
        xs[pl.ds(j * PCH, PCH), :] = lax.dot_general(
            selb, xb, (((0,), (0,)), ((), ())),
            preferred_element_type=jnp.float32)

    pltpu.make_async_copy(segv, seg_smem, ssem).wait()

    def expert_body(e, carry):
        slot = lax.rem(e, NBUF)
        pltpu.make_async_copy(g_hbm.at[e], gbuf.at[slot],
                              gsem.at[slot]).wait()
        pltpu.make_async_copy(l_hbm.at[e], lbuf.at[slot],
                              lsem.at[slot]).wait()
        start = seg_smem[0, e]
        end = seg_smem[1, e]
        tile0 = seg_smem[2, e]
        ntiles = seg_smem[3, e]
        sc = scale_ref[e]
        w0 = gbuf[slot, 0]
        w1 = gbuf[slot, 1]
        w2 = lbuf[slot]

        def tile_body(t, c):
            r0 = (tile0 + t) * ROW_TILE
            rows = xs[pl.ds(r0, ROW_TILE), :]
            dn = (((1,), (1,)), ((), ()))
            g0 = lax.dot_general(rows, w0, dn,
                                 preferred_element_type=jnp.float32)
            g1 = lax.dot_general(rows, w1, dn,
                                 preferred_element_type=jnp.float32)
            act = jax.nn.gelu(g0) * g1
            o = jnp.dot(act, w2, preferred_element_type=jnp.float32)
            ridx = r0 + lax.broadcasted_iota(jnp.int32, (ROW_TILE, 1), 0)
            m = (ridx >= start) & (ridx < end)
            out_ref[pl.ds(r0, ROW_TILE), :] = jnp.where(
                m, o * sc, out_ref[pl.ds(r0, ROW_TILE), :])
            return c

        lax.fori_loop(0, ntiles, tile_body, 0)

        @pl.when(e + NBUF < E)
        def _():
            start_fetch(e + NBUF, slot)

        return carry

    lax.fori_loop(0, E, expert_body, 0)


def _moe(x2d, router_scale, router_logits, gating, linear, scale):
    T, D = x2d.shape
    E, _, H, _ = gating.shape
    return pl.pallas_call(
        _moe_body,
        in_specs=[
            pl.BlockSpec(memory_space=pltpu.SMEM),
            pl.BlockSpec(memory_space=pltpu.VMEM),
            pl.BlockSpec(memory_space=pltpu.VMEM),
            pl.BlockSpec(memory_space=pl.ANY),
            pl.BlockSpec(memory_space=pl.ANY),
            pl.BlockSpec(memory_space=pl.ANY),
        ],
        out_specs=(
            pl.BlockSpec(memory_space=pltpu.VMEM),
            pl.BlockSpec(memory_space=pltpu.VMEM),
        ),
        out_shape=(
            jax.ShapeDtypeStruct((T, D), jnp.float32),
            jax.ShapeDtypeStruct((T,), jnp.int32),
        ),
        scratch_shapes=[
            pltpu.VMEM((T, D), jnp.float32),
            pltpu.VMEM((T, D), jnp.float32),
            pltpu.VMEM((4, E), jnp.int32),
            pltpu.VMEM((NBUF, 2, H, D), jnp.float32),
            pltpu.VMEM((NBUF, H, D), jnp.float32),
            pltpu.SMEM((4, E), jnp.int32),
            pltpu.SemaphoreType.DMA,
            pltpu.SemaphoreType.DMA,
            pltpu.SemaphoreType.DMA((NBUF,)),
            pltpu.SemaphoreType.DMA((NBUF,)),
        ],
    )(scale, router_scale, router_logits, x2d, gating, linear)


def kernel(x, router_scale, router_logits, gating_einsum, linear,
           per_expert_scale):
    B, L, D = x.shape
    x2d = x.reshape(B * L, D)
    out_sorted, pos = _moe(x2d, router_scale, router_logits,
                           gating_einsum, linear, per_expert_scale)
    out = _sc_gather_rows(out_sorted, pos)
    return out.reshape(B, L, D)
